# Initial kernel scaffold; baseline (speedup 1.0000x reference)
import jax
import jax.numpy as jnp
from jax.experimental import pallas as pl


def kernel(x, n1g, n1b, Wqkv, bqkv, rpb, Wproj, bproj, n2g, n2b, W1, b1, W2, b2):
    raise NotImplementedError("write your pallas kernel here")


# speedup vs baseline: 19.1109x; 19.1109x over previous
import functools

import jax
import jax.numpy as jnp
import numpy as np
from jax.experimental import pallas as pl

DIM = 384
HEADS = 12
HD = DIM // HEADS
WS = 7
H = 28
W = 28
B = 4
NW = 16
N = WS * WS
HIDDEN = 1536
TOK = B * H * W
SCALE = HD ** -0.5


def _rel_pos_index(ws):
    coords = np.stack(np.meshgrid(np.arange(ws), np.arange(ws), indexing='ij'))
    cf = coords.reshape(2, -1)
    rel = cf[:, :, None] - cf[:, None, :]
    rel = rel.transpose(1, 2, 0).copy()
    rel[:, :, 0] += ws - 1
    rel[:, :, 1] += ws - 1
    rel[:, :, 0] *= 2 * ws - 1
    return rel.sum(-1)

_RPI_FLAT = np.asarray(_rel_pos_index(WS).reshape(-1), dtype=np.int32)


def _qkv_body(x_ref, g_ref, b_ref, w_ref, bias_ref, o_ref):
    x = x_ref[...]
    mu = jnp.mean(x, axis=1, keepdims=True)
    var = jnp.mean((x - mu) ** 2, axis=1, keepdims=True)
    ln = (x - mu) * jax.lax.rsqrt(var + 1e-5) * g_ref[...] + b_ref[...]
    o_ref[...] = jax.lax.dot_general(
        ln, w_ref[...], (((1,), (1,)), ((), ())),
        preferred_element_type=jnp.float32) + bias_ref[...]


def _attn_body(q_ref, k_ref, v_ref, bias_ref, o_ref):
    q = q_ref[0]
    k = k_ref[0]
    v = v_ref[0]
    s = jax.lax.dot_general(
        q, k, (((2,), (2,)), ((0,), (0,))),
        preferred_element_type=jnp.float32) * SCALE
    s = s + bias_ref[...]
    m = jnp.max(s, axis=-1, keepdims=True)
    e = jnp.exp(s - m)
    p = e / jnp.sum(e, axis=-1, keepdims=True)
    o = jax.lax.dot_general(
        p, v, (((2,), (1,)), ((0,), (0,))),
        preferred_element_type=jnp.float32)
    o_ref[0] = (o[:, 0:196] + o[:, 196:392]
                + o[:, 392:588] + o[:, 588:784]) * 0.25


def _mlp_body(a_ref, x_ref, wp_ref, bp_ref, g2_ref, be2_ref,
              w1_ref, b1_ref, w2_ref, b2_ref, o_ref):
    z = jax.lax.dot_general(
        a_ref[...], wp_ref[...], (((1,), (1,)), ((), ())),
        preferred_element_type=jnp.float32) + bp_ref[...]
    x2 = x_ref[...] + z
    mu = jnp.mean(x2, axis=1, keepdims=True)
    var = jnp.mean((x2 - mu) ** 2, axis=1, keepdims=True)
    ln = (x2 - mu) * jax.lax.rsqrt(var + 1e-5) * g2_ref[...] + be2_ref[...]
    h1 = jax.lax.dot_general(
        ln, w1_ref[...], (((1,), (1,)), ((), ())),
        preferred_element_type=jnp.float32) + b1_ref[...]
    h1 = 0.5 * h1 * (1.0 + jax.lax.erf(h1 * (2.0 ** -0.5)))
    y = jax.lax.dot_general(
        h1, w2_ref[...], (((1,), (1,)), ((), ())),
        preferred_element_type=jnp.float32) + b2_ref[...]
    o_ref[...] = x2 + y


def kernel(x, n1g, n1b, Wqkv, bqkv, rpb, Wproj, bproj, n2g, n2b, W1, b1, W2, b2):
    xw = (x.reshape(B, H // WS, WS, W // WS, WS, DIM)
           .transpose(0, 1, 3, 2, 4, 5)
           .reshape(TOK, DIM))

    qkv_tile = 448
    qkv = pl.pallas_call(
        _qkv_body,
        grid=(TOK // qkv_tile,),
        in_specs=[
            pl.BlockSpec((qkv_tile, DIM), lambda i: (i, 0)),
            pl.BlockSpec((1, DIM), lambda i: (0, 0)),
            pl.BlockSpec((1, DIM), lambda i: (0, 0)),
            pl.BlockSpec((3 * DIM, DIM), lambda i: (0, 0)),
            pl.BlockSpec((1, 3 * DIM), lambda i: (0, 0)),
        ],
        out_specs=pl.BlockSpec((qkv_tile, 3 * DIM), lambda i: (i, 0)),
        out_shape=jax.ShapeDtypeStruct((TOK, 3 * DIM), jnp.float32),
    )(xw, n1g.reshape(1, DIM), n1b.reshape(1, DIM), Wqkv, bqkv.reshape(1, 3 * DIM))

    q = qkv[:, :DIM].reshape(4, 4, 4, N, HEADS, HD)
    qs = q.transpose(2, 4, 0, 1, 3, 5).reshape(4, HEADS, 16 * N, HD)
    k16 = qkv[:NW * N, DIM:2 * DIM].reshape(NW, N, HEADS, HD).transpose(0, 2, 1, 3)
    v16 = qkv[:NW * N, 2 * DIM:].reshape(NW, N, HEADS, HD).transpose(0, 2, 1, 3)

    bias = rpb[_RPI_FLAT].reshape(N, N, HEADS).transpose(2, 0, 1)
    bias_big = jnp.broadcast_to(bias[:, None], (HEADS, 16, N, N)).reshape(
        HEADS, 16 * N, N)

    att = pl.pallas_call(
        _attn_body,
        grid=(NW,),
        in_specs=[
            pl.BlockSpec((1, HEADS, 16 * N, HD), lambda w: (w // 4, 0, 0, 0)),
            pl.BlockSpec((1, HEADS, N, HD), lambda w: (w, 0, 0, 0)),
            pl.BlockSpec((1, HEADS, N, HD), lambda w: (w, 0, 0, 0)),
            pl.BlockSpec((HEADS, 16 * N, N), lambda w: (0, 0, 0)),
        ],
        out_specs=pl.BlockSpec((1, HEADS, 4 * N, HD), lambda w: (w, 0, 0, 0)),
        out_shape=jax.ShapeDtypeStruct((NW, HEADS, 4 * N, HD), jnp.float32),
    )(qs, k16, v16, bias_big)

    attw = (att.reshape(NW, HEADS, 4, N, HD)
               .transpose(2, 0, 3, 1, 4)
               .reshape(B, H // WS, W // WS, WS, WS, DIM)
               .transpose(0, 1, 3, 2, 4, 5)
               .reshape(TOK, DIM))

    mlp_tile = 392
    out = pl.pallas_call(
        _mlp_body,
        grid=(TOK // mlp_tile,),
        in_specs=[
            pl.BlockSpec((mlp_tile, DIM), lambda i: (i, 0)),
            pl.BlockSpec((mlp_tile, DIM), lambda i: (i, 0)),
            pl.BlockSpec((DIM, DIM), lambda i: (0, 0)),
            pl.BlockSpec((1, DIM), lambda i: (0, 0)),
            pl.BlockSpec((1, DIM), lambda i: (0, 0)),
            pl.BlockSpec((1, DIM), lambda i: (0, 0)),
            pl.BlockSpec((HIDDEN, DIM), lambda i: (0, 0)),
            pl.BlockSpec((1, HIDDEN), lambda i: (0, 0)),
            pl.BlockSpec((DIM, HIDDEN), lambda i: (0, 0)),
            pl.BlockSpec((1, DIM), lambda i: (0, 0)),
        ],
        out_specs=pl.BlockSpec((mlp_tile, DIM), lambda i: (i, 0)),
        out_shape=jax.ShapeDtypeStruct((TOK, DIM), jnp.float32),
    )(attw, x.reshape(TOK, DIM), Wproj, bproj.reshape(1, DIM),
      n2g.reshape(1, DIM), n2b.reshape(1, DIM),
      W1, b1.reshape(1, HIDDEN), W2, b2.reshape(1, DIM))

    return out.reshape(B, H * W, DIM)


# speedup vs baseline: 26.0321x; 1.3622x over previous
import jax
import jax.numpy as jnp
import numpy as np
from jax.experimental import pallas as pl

DIM = 384
HEADS = 12
HD = DIM // HEADS
WS = 7
H = 28
W = 28
B = 4
NW = 16
N = WS * WS
NP = 56
HIDDEN = 1536
TOKP = B * NW * NP
SCALE = HD ** -0.5


def _rel_pos_index(ws):
    coords = np.stack(np.meshgrid(np.arange(ws), np.arange(ws), indexing='ij'))
    cf = coords.reshape(2, -1)
    rel = cf[:, :, None] - cf[:, None, :]
    rel = rel.transpose(1, 2, 0).copy()
    rel[:, :, 0] += ws - 1
    rel[:, :, 1] += ws - 1
    rel[:, :, 0] *= 2 * ws - 1
    return rel.sum(-1)

_RPI_FLAT = np.asarray(_rel_pos_index(WS).reshape(-1), dtype=np.int32)


def _qkv_body(x_ref, g_ref, b_ref, w_ref, bias_ref, o_ref):
    x = x_ref[...]
    mu = jnp.mean(x, axis=1, keepdims=True)
    var = jnp.mean((x - mu) ** 2, axis=1, keepdims=True)
    ln = (x - mu) * jax.lax.rsqrt(var + 1e-5) * g_ref[...] + b_ref[...]
    res = jax.lax.dot_general(
        ln, w_ref[...], (((1,), (1,)), ((), ())),
        preferred_element_type=jnp.float32) + bias_ref[...]
    o_ref[0] = res[:, :DIM] * SCALE
    o_ref[1] = res[:, DIM:2 * DIM]
    o_ref[2] = res[:, 2 * DIM:]


def _attn_body(q_ref, k_ref, v_ref, bias_ref, o_ref):
    qf = q_ref[0].reshape(16 * NP, DIM)
    kf = k_ref[0, 0]
    vf = v_ref[0, 0]
    rows = jax.lax.broadcasted_iota(jnp.int32, (16 * NP, NP), 0)
    lanes = jax.lax.broadcasted_iota(jnp.int32, (16 * NP, NP), 1)
    onehot = (jax.lax.rem(rows, NP) == lanes).astype(jnp.float32)
    keymask = lanes >= N
    for h in range(HEADS):
        sl = slice(h * HD, (h + 1) * HD)
        a = jnp.concatenate([qf[:, sl], onehot], axis=1)
        bmat = jnp.concatenate([kf[:, sl], bias_ref[h]], axis=1)
        s = jax.lax.dot_general(
            a, bmat, (((1,), (1,)), ((), ())),
            preferred_element_type=jnp.float32)
        s = jnp.where(keymask, -1e30, s)
        m = jnp.max(s, axis=-1, keepdims=True)
        e = jnp.exp(s - m)
        p = e / jnp.sum(e, axis=-1, keepdims=True)
        o = jax.lax.dot_general(
            p, vf[:, sl], (((1,), (0,)), ((), ())),
            preferred_element_type=jnp.float32)
        o4 = (o[0:224] + o[224:448] + o[448:672] + o[672:896]) * 0.25
        o_ref[:, 0, :, sl] = o4.reshape(4, NP, HD)


def _mlp_body(a_ref, x_ref, wp_ref, bp_ref, g2_ref, be2_ref,
              w1_ref, b1_ref, w2_ref, b2_ref, o_ref):
    z = jax.lax.dot_general(
        a_ref[...], wp_ref[...], (((1,), (1,)), ((), ())),
        preferred_element_type=jnp.float32) + bp_ref[...]
    x2 = x_ref[...] + z
    mu = jnp.mean(x2, axis=1, keepdims=True)
    var = jnp.mean((x2 - mu) ** 2, axis=1, keepdims=True)
    ln = (x2 - mu) * jax.lax.rsqrt(var + 1e-5) * g2_ref[...] + be2_ref[...]
    h1 = jax.lax.dot_general(
        ln, w1_ref[...], (((1,), (1,)), ((), ())),
        preferred_element_type=jnp.float32) + b1_ref[...]
    h1 = 0.5 * h1 * (1.0 + jax.lax.erf(h1 * (2.0 ** -0.5)))
    y = jax.lax.dot_general(
        h1, w2_ref[...], (((1,), (1,)), ((), ())),
        preferred_element_type=jnp.float32) + b2_ref[...]
    o_ref[...] = x2 + y


def kernel(x, n1g, n1b, Wqkv, bqkv, rpb, Wproj, bproj, n2g, n2b, W1, b1, W2, b2):
    f32 = jnp.float32
    xw = (x.reshape(B, H // WS, WS, W // WS, WS, DIM)
           .transpose(0, 1, 3, 2, 4, 5)
           .reshape(B * NW, N, DIM))
    xw = jnp.pad(xw, ((0, 0), (0, NP - N), (0, 0))).reshape(TOKP, DIM)

    t1 = 448
    qkv3 = pl.pallas_call(
        _qkv_body,
        grid=(TOKP // t1,),
        in_specs=[
            pl.BlockSpec((t1, DIM), lambda i: (i, 0)),
            pl.BlockSpec((1, DIM), lambda i: (0, 0)),
            pl.BlockSpec((1, DIM), lambda i: (0, 0)),
            pl.BlockSpec((3 * DIM, DIM), lambda i: (0, 0)),
            pl.BlockSpec((1, 3 * DIM), lambda i: (0, 0)),
        ],
        out_specs=pl.BlockSpec((3, t1, DIM), lambda i: (0, i, 0)),
        out_shape=jax.ShapeDtypeStruct((3, TOKP, DIM), f32),
    )(xw, n1g.reshape(1, DIM), n1b.reshape(1, DIM), Wqkv, bqkv.reshape(1, 3 * DIM))

    q6 = qkv3.reshape(3, 4, 4, 4, NP, DIM)
    kv4 = qkv3.reshape(3, B * NW, NP, DIM)

    rpbg = rpb[_RPI_FLAT].reshape(N, N, HEADS)
    biasT = jnp.zeros((HEADS, NP, NP), f32).at[:, :N, :N].set(
        rpbg.transpose(2, 1, 0))

    att = pl.pallas_call(
        _attn_body,
        grid=(NW,),
        in_specs=[
            pl.BlockSpec((1, 4, 4, 1, NP, DIM), lambda w: (0, 0, 0, w // 4, 0, 0)),
            pl.BlockSpec((1, 1, NP, DIM), lambda w: (1, w, 0, 0)),
            pl.BlockSpec((1, 1, NP, DIM), lambda w: (2, w, 0, 0)),
            pl.BlockSpec((HEADS, NP, NP), lambda w: (0, 0, 0)),
        ],
        out_specs=pl.BlockSpec((4, 1, NP, DIM), lambda w: (0, w, 0, 0)),
        out_shape=jax.ShapeDtypeStruct((B, NW, NP, DIM), f32),
    )(q6, kv4, kv4, biasT)

    attf = att.reshape(TOKP, DIM)

    t3 = 448
    out = pl.pallas_call(
        _mlp_body,
        grid=(TOKP // t3,),
        in_specs=[
            pl.BlockSpec((t3, DIM), lambda i: (i, 0)),
            pl.BlockSpec((t3, DIM), lambda i: (i, 0)),
            pl.BlockSpec((DIM, DIM), lambda i: (0, 0)),
            pl.BlockSpec((1, DIM), lambda i: (0, 0)),
            pl.BlockSpec((1, DIM), lambda i: (0, 0)),
            pl.BlockSpec((1, DIM), lambda i: (0, 0)),
            pl.BlockSpec((HIDDEN, DIM), lambda i: (0, 0)),
            pl.BlockSpec((1, HIDDEN), lambda i: (0, 0)),
            pl.BlockSpec((DIM, HIDDEN), lambda i: (0, 0)),
            pl.BlockSpec((1, DIM), lambda i: (0, 0)),
        ],
        out_specs=pl.BlockSpec((t3, DIM), lambda i: (i, 0)),
        out_shape=jax.ShapeDtypeStruct((TOKP, DIM), f32),
    )(attf, xw, Wproj, bproj.reshape(1, DIM),
      n2g.reshape(1, DIM), n2b.reshape(1, DIM),
      W1, b1.reshape(1, HIDDEN), W2, b2.reshape(1, DIM))

    y = out.reshape(B * NW, NP, DIM)[:, :N]
    y = (y.reshape(B, H // WS, W // WS, WS, WS, DIM)
          .transpose(0, 1, 3, 2, 4, 5)
          .reshape(B, H * W, DIM))
    return y


# speedup vs baseline: 39.1258x; 1.5030x over previous
import jax
import jax.numpy as jnp
import numpy as np
from jax.experimental import pallas as pl
from jax.experimental.pallas import tpu as pltpu

DIM = 384
HEADS = 12
HD = DIM // HEADS
WS = 7
H = 28
W = 28
B = 4
NW = 16
N = WS * WS
NP = 56
HIDDEN = 1536
TOK = B * H * W
TOKP = B * NW * NP
SCALE = HD ** -0.5


def _rel_pos_index(ws):
    coords = np.stack(np.meshgrid(np.arange(ws), np.arange(ws), indexing='ij'))
    cf = coords.reshape(2, -1)
    rel = cf[:, :, None] - cf[:, None, :]
    rel = rel.transpose(1, 2, 0).copy()
    rel[:, :, 0] += ws - 1
    rel[:, :, 1] += ws - 1
    rel[:, :, 0] *= 2 * ws - 1
    return rel.sum(-1)

_RPI_FLAT = np.asarray(_rel_pos_index(WS).reshape(-1), dtype=np.int32)


def _qkv_body(x_ref, g_ref, b_ref, w_ref, bias_ref, o_ref, xwb):
    for wl in range(2):
        for wc in range(4):
            for i in range(WS):
                dst = (wl * 4 + wc) * NP + i * WS
                src = wl * 196 + i * 28 + wc * WS
                xwb[dst:dst + WS, :] = x_ref[src:src + WS, :]
            xwb[(wl * 4 + wc) * NP + N:(wl * 4 + wc) * NP + NP, :] = jnp.zeros(
                (NP - N, DIM), jnp.float32)
    x = xwb[...]
    mu = jnp.mean(x, axis=1, keepdims=True)
    var = jnp.mean((x - mu) ** 2, axis=1, keepdims=True)
    ln = (x - mu) * jax.lax.rsqrt(var + 1e-5) * g_ref[...] + b_ref[...]
    res = jax.lax.dot_general(
        ln, w_ref[...], (((1,), (1,)), ((), ())),
        preferred_element_type=jnp.float32) + bias_ref[...]
    o_ref[0] = res[:, :DIM] * SCALE
    o_ref[1] = res[:, DIM:2 * DIM]
    o_ref[2] = res[:, 2 * DIM:]


def _attn_body(q_ref, k_ref, v_ref, bias_ref, o_ref):
    qf = q_ref[0].reshape(16 * NP, DIM)
    kf = k_ref[0, 0]
    vf = v_ref[0, 0]
    lanes2 = jax.lax.broadcasted_iota(jnp.int32, (16 * NP, 2 * NP), 1)
    laneso = jax.lax.broadcasted_iota(jnp.int32, (16 * NP, 2 * HD), 1)
    z56 = jnp.zeros((NP, HD), jnp.float32)
    for i in range(HEADS // 2):
        sl = slice(i * 2 * HD, (i + 1) * 2 * HD)
        k1 = kf[:, i * 2 * HD:i * 2 * HD + HD]
        k2 = kf[:, i * 2 * HD + HD:(i + 1) * 2 * HD]
        bd_k = jnp.concatenate([
            jnp.concatenate([k1, z56], axis=1),
            jnp.concatenate([z56, k2], axis=1)], axis=0)
        s = jax.lax.dot_general(
            qf[:, sl], bd_k, (((1,), (1,)), ((), ())),
            preferred_element_type=jnp.float32) + bias_ref[i]
        m1 = jnp.max(s[:, :NP], axis=-1, keepdims=True)
        m2 = jnp.max(s[:, NP:], axis=-1, keepdims=True)
        e = jnp.exp(s - jnp.where(lanes2 < NP, m1, m2))
        s1 = jnp.sum(e[:, :NP], axis=-1, keepdims=True)
        s2 = jnp.sum(e[:, NP:], axis=-1, keepdims=True)
        v1 = vf[:, i * 2 * HD:i * 2 * HD + HD]
        v2 = vf[:, i * 2 * HD + HD:(i + 1) * 2 * HD]
        bd_v = jnp.concatenate([
            jnp.concatenate([v1, z56], axis=1),
            jnp.concatenate([z56, v2], axis=1)], axis=0)
        o = jax.lax.dot_general(
            e, bd_v, (((1,), (0,)), ((), ())),
            preferred_element_type=jnp.float32)
        o = o * jnp.where(laneso < HD, 1.0 / s1, 1.0 / s2)
        o4 = (o[0:224] + o[224:448] + o[448:672] + o[672:896]) * 0.25
        o_ref[:, 0, :, sl] = o4.reshape(4, NP, 2 * HD)


def _mlp_body(a_ref, x_ref, wp_ref, bp_ref, g2_ref, be2_ref,
              w1_ref, b1_ref, w2_ref, b2_ref, o_ref, awb):
    for wl in range(2):
        for wc in range(4):
            for i in range(WS):
                src = (wl * 4 + wc) * NP + i * WS
                dst = wl * 196 + i * 28 + wc * WS
                awb[dst:dst + WS, :] = a_ref[src:src + WS, :]
    z = jax.lax.dot_general(
        awb[...], wp_ref[...], (((1,), (1,)), ((), ())),
        preferred_element_type=jnp.float32) + bp_ref[...]
    x2 = x_ref[...] + z
    mu = jnp.mean(x2, axis=1, keepdims=True)
    var = jnp.mean((x2 - mu) ** 2, axis=1, keepdims=True)
    ln = (x2 - mu) * jax.lax.rsqrt(var + 1e-5) * g2_ref[...] + be2_ref[...]
    h1 = jax.lax.dot_general(
        ln, w1_ref[...], (((1,), (1,)), ((), ())),
        preferred_element_type=jnp.float32) + b1_ref[...]
    h1 = 0.5 * h1 * (1.0 + jax.lax.erf(h1 * (2.0 ** -0.5)))
    y = jax.lax.dot_general(
        h1, w2_ref[...], (((1,), (1,)), ((), ())),
        preferred_element_type=jnp.float32) + b2_ref[...]
    o_ref[...] = x2 + y


def kernel(x, n1g, n1b, Wqkv, bqkv, rpb, Wproj, bproj, n2g, n2b, W1, b1, W2, b2):
    f32 = jnp.float32
    xf = x.reshape(TOK, DIM)

    qkv3 = pl.pallas_call(
        _qkv_body,
        grid=(8,),
        in_specs=[
            pl.BlockSpec((392, DIM), lambda i: (i, 0)),
            pl.BlockSpec((1, DIM), lambda i: (0, 0)),
            pl.BlockSpec((1, DIM), lambda i: (0, 0)),
            pl.BlockSpec((3 * DIM, DIM), lambda i: (0, 0)),
            pl.BlockSpec((1, 3 * DIM), lambda i: (0, 0)),
        ],
        out_specs=pl.BlockSpec((3, 448, DIM), lambda i: (0, i, 0)),
        out_shape=jax.ShapeDtypeStruct((3, TOKP, DIM), f32),
        scratch_shapes=[pltpu.VMEM((448, DIM), f32)],
    )(xf, n1g.reshape(1, DIM), n1b.reshape(1, DIM), Wqkv, bqkv.reshape(1, 3 * DIM))

    q6 = qkv3.reshape(3, 4, 4, 4, NP, DIM)
    kv4 = qkv3.reshape(3, B * NW, NP, DIM)

    rpbg = rpb[_RPI_FLAT].reshape(N, N, HEADS)
    full = jnp.full((HEADS, NP, NP), -1e30, f32).at[:, :N, :N].set(
        rpbg.transpose(2, 0, 1))
    tiled = jnp.broadcast_to(full[:, None], (HEADS, 16, NP, NP)).reshape(
        HEADS, 16 * NP, NP)
    bias2 = jnp.concatenate([tiled[0::2], tiled[1::2]], axis=2)

    att = pl.pallas_call(
        _attn_body,
        grid=(NW,),
        in_specs=[
            pl.BlockSpec((1, 4, 4, 1, NP, DIM), lambda w: (0, 0, 0, w // 4, 0, 0)),
            pl.BlockSpec((1, 1, NP, DIM), lambda w: (1, w, 0, 0)),
            pl.BlockSpec((1, 1, NP, DIM), lambda w: (2, w, 0, 0)),
            pl.BlockSpec((HEADS // 2, 16 * NP, 2 * NP), lambda w: (0, 0, 0)),
        ],
        out_specs=pl.BlockSpec((4, 1, NP, DIM), lambda w: (0, w, 0, 0)),
        out_shape=jax.ShapeDtypeStruct((B, NW, NP, DIM), f32),
    )(q6, kv4, kv4, bias2)

    attf = att.reshape(TOKP, DIM)

    out = pl.pallas_call(
        _mlp_body,
        grid=(8,),
        in_specs=[
            pl.BlockSpec((448, DIM), lambda i: (i, 0)),
            pl.BlockSpec((392, DIM), lambda i: (i, 0)),
            pl.BlockSpec((DIM, DIM), lambda i: (0, 0)),
            pl.BlockSpec((1, DIM), lambda i: (0, 0)),
            pl.BlockSpec((1, DIM), lambda i: (0, 0)),
            pl.BlockSpec((1, DIM), lambda i: (0, 0)),
            pl.BlockSpec((HIDDEN, DIM), lambda i: (0, 0)),
            pl.BlockSpec((1, HIDDEN), lambda i: (0, 0)),
            pl.BlockSpec((DIM, HIDDEN), lambda i: (0, 0)),
            pl.BlockSpec((1, DIM), lambda i: (0, 0)),
        ],
        out_specs=pl.BlockSpec((392, DIM), lambda i: (i, 0)),
        out_shape=jax.ShapeDtypeStruct((TOK, DIM), f32),
        scratch_shapes=[pltpu.VMEM((392, DIM), f32)],
    )(attf, xf, Wproj, bproj.reshape(1, DIM),
      n2g.reshape(1, DIM), n2b.reshape(1, DIM),
      W1, b1.reshape(1, HIDDEN), W2, b2.reshape(1, DIM))

    return out.reshape(B, H * W, DIM)


# speedup vs baseline: 42.1217x; 1.0766x over previous
import jax
import jax.numpy as jnp
import numpy as np
from jax.experimental import pallas as pl
from jax.experimental.pallas import tpu as pltpu

DIM = 384
HEADS = 12
HD = DIM // HEADS
WS = 7
H = 28
W = 28
B = 4
NW = 16
N = WS * WS
NP = 56
HIDDEN = 1536
TOK = B * H * W
TOKP = B * NW * NP
SCALE = HD ** -0.5


def _rel_pos_index(ws):
    coords = np.stack(np.meshgrid(np.arange(ws), np.arange(ws), indexing='ij'))
    cf = coords.reshape(2, -1)
    rel = cf[:, :, None] - cf[:, None, :]
    rel = rel.transpose(1, 2, 0).copy()
    rel[:, :, 0] += ws - 1
    rel[:, :, 1] += ws - 1
    rel[:, :, 0] *= 2 * ws - 1
    return rel.sum(-1)

_RPI_FLAT = np.asarray(_rel_pos_index(WS).reshape(-1), dtype=np.int32)


def _qkv_body(x_ref, g_ref, b_ref, w_ref, bias_ref, o_ref, xwb):
    for wl in range(2):
        for wc in range(4):
            for i in range(WS):
                dst = (wl * 4 + wc) * NP + i * WS
                src = wl * 196 + i * 28 + wc * WS
                xwb[dst:dst + WS, :] = x_ref[src:src + WS, :]
            xwb[(wl * 4 + wc) * NP + N:(wl * 4 + wc) * NP + NP, :] = jnp.zeros(
                (NP - N, DIM), jnp.float32)
    x = xwb[...]
    mu = jnp.mean(x, axis=1, keepdims=True)
    var = jnp.mean((x - mu) ** 2, axis=1, keepdims=True)
    ln = (x - mu) * jax.lax.rsqrt(var + 1e-5) * g_ref[...] + b_ref[...]
    res = jax.lax.dot_general(
        ln, w_ref[...], (((1,), (1,)), ((), ())),
        preferred_element_type=jnp.float32) + bias_ref[...]
    o_ref[0] = res[:, :DIM] * SCALE
    o_ref[1] = res[:, DIM:2 * DIM]
    o_ref[2] = res[:, 2 * DIM:]


def _attn_body(q_ref, k_ref, v_ref, bias_ref, o_ref):
    f32 = jnp.float32
    qf = q_ref[0].reshape(16 * NP, DIM)
    lanes2 = jax.lax.broadcasted_iota(jnp.int32, (16 * NP, 2 * NP), 1)
    laneso = jax.lax.broadcasted_iota(jnp.int32, (16 * NP, 2 * HD), 1)
    z56 = jnp.zeros((NP, HD), f32)
    o1 = jnp.ones((NP, 1), f32)
    zz1 = jnp.zeros((NP, 1), f32)
    sumcols = jnp.concatenate([
        jnp.concatenate([o1, zz1], axis=1),
        jnp.concatenate([zz1, o1], axis=1)], axis=0)
    for b in range(4):
        kf = k_ref[0, b]
        vf = v_ref[0, b]
        for i in range(HEADS // 2):
            sl = slice(i * 2 * HD, (i + 1) * 2 * HD)
            k1 = kf[:, i * 2 * HD:i * 2 * HD + HD]
            k2 = kf[:, i * 2 * HD + HD:(i + 1) * 2 * HD]
            bd_k = jnp.concatenate([
                jnp.concatenate([k1, z56], axis=1),
                jnp.concatenate([z56, k2], axis=1)], axis=0)
            s = jax.lax.dot_general(
                qf[:, sl], bd_k, (((1,), (1,)), ((), ())),
                preferred_element_type=f32)
            s = (s.reshape(16, NP, 2 * NP) + bias_ref[i][None]
                 ).reshape(16 * NP, 2 * NP)
            m1 = jnp.max(s[:, :NP], axis=-1, keepdims=True)
            m2 = jnp.max(s[:, NP:], axis=-1, keepdims=True)
            e = jnp.exp(s - jnp.where(lanes2 < NP, m1, m2))
            v1 = vf[:, i * 2 * HD:i * 2 * HD + HD]
            v2 = vf[:, i * 2 * HD + HD:(i + 1) * 2 * HD]
            bd_v = jnp.concatenate([
                jnp.concatenate([v1, z56, sumcols[:NP]], axis=1),
                jnp.concatenate([z56, v2, sumcols[NP:]], axis=1)], axis=0)
            oa = jax.lax.dot_general(
                e, bd_v, (((1,), (0,)), ((), ())),
                preferred_element_type=f32)
            rs1 = 1.0 / oa[:, 2 * HD:2 * HD + 1]
            rs2 = 1.0 / oa[:, 2 * HD + 1:2 * HD + 2]
            o = oa[:, :2 * HD] * jnp.where(laneso < HD, rs1, rs2)
            o4 = (o[0:224] + o[224:448] + o[448:672] + o[672:896]) * 0.25
            o_ref[:, b, :, sl] = o4.reshape(4, NP, 2 * HD)


def _mlp_body(a_ref, x_ref, wp_ref, bp_ref, g2_ref, be2_ref,
              w1_ref, b1_ref, w2_ref, b2_ref, o_ref, awb):
    for wl in range(2):
        for wc in range(4):
            for i in range(WS):
                src = (wl * 4 + wc) * NP + i * WS
                dst = wl * 196 + i * 28 + wc * WS
                awb[dst:dst + WS, :] = a_ref[src:src + WS, :]
    z = jax.lax.dot_general(
        awb[...], wp_ref[...], (((1,), (1,)), ((), ())),
        preferred_element_type=jnp.float32) + bp_ref[...]
    x2 = x_ref[...] + z
    mu = jnp.mean(x2, axis=1, keepdims=True)
    var = jnp.mean((x2 - mu) ** 2, axis=1, keepdims=True)
    ln = (x2 - mu) * jax.lax.rsqrt(var + 1e-5) * g2_ref[...] + be2_ref[...]
    h1 = jax.lax.dot_general(
        ln, w1_ref[...], (((1,), (1,)), ((), ())),
        preferred_element_type=jnp.float32) + b1_ref[...]
    h1 = 0.5 * h1 * (1.0 + jax.lax.erf(h1 * (2.0 ** -0.5)))
    y = jax.lax.dot_general(
        h1, w2_ref[...], (((1,), (1,)), ((), ())),
        preferred_element_type=jnp.float32) + b2_ref[...]
    o_ref[...] = x2 + y


def kernel(x, n1g, n1b, Wqkv, bqkv, rpb, Wproj, bproj, n2g, n2b, W1, b1, W2, b2):
    f32 = jnp.float32
    xf = x.reshape(TOK, DIM)

    qkv3 = pl.pallas_call(
        _qkv_body,
        grid=(8,),
        in_specs=[
            pl.BlockSpec((392, DIM), lambda i: (i, 0)),
            pl.BlockSpec((1, DIM), lambda i: (0, 0)),
            pl.BlockSpec((1, DIM), lambda i: (0, 0)),
            pl.BlockSpec((3 * DIM, DIM), lambda i: (0, 0)),
            pl.BlockSpec((1, 3 * DIM), lambda i: (0, 0)),
        ],
        out_specs=pl.BlockSpec((3, 448, DIM), lambda i: (0, i, 0)),
        out_shape=jax.ShapeDtypeStruct((3, TOKP, DIM), f32),
        scratch_shapes=[pltpu.VMEM((448, DIM), f32)],
    )(xf, n1g.reshape(1, DIM), n1b.reshape(1, DIM), Wqkv, bqkv.reshape(1, 3 * DIM))

    q6 = qkv3.reshape(3, 4, 4, 4, NP, DIM)
    kv4 = qkv3.reshape(3, B * NW, NP, DIM)

    rpbg = rpb[_RPI_FLAT].reshape(N, N, HEADS)
    full = jnp.full((HEADS, NP, NP), -1e30, f32).at[:, :N, :N].set(
        rpbg.transpose(2, 0, 1))
    bias2 = jnp.concatenate([full[0::2], full[1::2]], axis=2)

    att = pl.pallas_call(
        _attn_body,
        grid=(4,),
        in_specs=[
            pl.BlockSpec((1, 4, 4, 1, NP, DIM), lambda r: (0, 0, 0, r, 0, 0)),
            pl.BlockSpec((1, 4, NP, DIM), lambda r: (1, r, 0, 0)),
            pl.BlockSpec((1, 4, NP, DIM), lambda r: (2, r, 0, 0)),
            pl.BlockSpec((HEADS // 2, NP, 2 * NP), lambda r: (0, 0, 0)),
        ],
        out_specs=pl.BlockSpec((4, 4, NP, DIM), lambda r: (0, r, 0, 0)),
        out_shape=jax.ShapeDtypeStruct((B, NW, NP, DIM), f32),
    )(q6, kv4, kv4, bias2)

    attf = att.reshape(TOKP, DIM)

    out = pl.pallas_call(
        _mlp_body,
        grid=(8,),
        in_specs=[
            pl.BlockSpec((448, DIM), lambda i: (i, 0)),
            pl.BlockSpec((392, DIM), lambda i: (i, 0)),
            pl.BlockSpec((DIM, DIM), lambda i: (0, 0)),
            pl.BlockSpec((1, DIM), lambda i: (0, 0)),
            pl.BlockSpec((1, DIM), lambda i: (0, 0)),
            pl.BlockSpec((1, DIM), lambda i: (0, 0)),
            pl.BlockSpec((HIDDEN, DIM), lambda i: (0, 0)),
            pl.BlockSpec((1, HIDDEN), lambda i: (0, 0)),
            pl.BlockSpec((DIM, HIDDEN), lambda i: (0, 0)),
            pl.BlockSpec((1, DIM), lambda i: (0, 0)),
        ],
        out_specs=pl.BlockSpec((392, DIM), lambda i: (i, 0)),
        out_shape=jax.ShapeDtypeStruct((TOK, DIM), f32),
        scratch_shapes=[pltpu.VMEM((392, DIM), f32)],
    )(attf, xf, Wproj, bproj.reshape(1, DIM),
      n2g.reshape(1, DIM), n2b.reshape(1, DIM),
      W1, b1.reshape(1, HIDDEN), W2, b2.reshape(1, DIM))

    return out.reshape(B, H * W, DIM)


# speedup vs baseline: 45.6415x; 1.0836x over previous
import jax
import jax.numpy as jnp
import numpy as np
from jax.experimental import pallas as pl
from jax.experimental.pallas import tpu as pltpu

DIM = 384
HEADS = 12
HD = DIM // HEADS
WS = 7
H = 28
W = 28
B = 4
NW = 16
N = WS * WS
NP = 56
HIDDEN = 1536
TOK = B * H * W
TOKP = B * NW * NP
SCALE = HD ** -0.5


def _rel_pos_index(ws):
    coords = np.stack(np.meshgrid(np.arange(ws), np.arange(ws), indexing='ij'))
    cf = coords.reshape(2, -1)
    rel = cf[:, :, None] - cf[:, None, :]
    rel = rel.transpose(1, 2, 0).copy()
    rel[:, :, 0] += ws - 1
    rel[:, :, 1] += ws - 1
    rel[:, :, 0] *= 2 * ws - 1
    return rel.sum(-1)

_RPI_FLAT = np.asarray(_rel_pos_index(WS).reshape(-1), dtype=np.int32)


def _qkv_body(x_ref, g_ref, b_ref, w_ref, bias_ref, o_ref, xwb):
    for wl in range(2):
        for wc in range(4):
            for i in range(WS):
                dst = (wl * 4 + wc) * NP + i * WS
                src = wl * 196 + i * 28 + wc * WS
                xwb[dst:dst + WS, :] = x_ref[src:src + WS, :]
            xwb[(wl * 4 + wc) * NP + N:(wl * 4 + wc) * NP + NP, :] = jnp.zeros(
                (NP - N, DIM), jnp.float32)
    x = xwb[...]
    mu = jnp.mean(x, axis=1, keepdims=True)
    var = jnp.mean((x - mu) ** 2, axis=1, keepdims=True)
    ln = (x - mu) * jax.lax.rsqrt(var + 1e-5) * g_ref[...] + b_ref[...]
    res = jax.lax.dot_general(
        ln, w_ref[...], (((1,), (1,)), ((), ())),
        preferred_element_type=jnp.float32) + bias_ref[...]
    o_ref[0] = res[:, :DIM] * SCALE
    o_ref[1] = res[:, DIM:2 * DIM]
    o_ref[2] = res[:, 2 * DIM:]


def _attn_body(q_ref, k_ref, v_ref, bias_ref, o_ref):
    f32 = jnp.float32
    qf = q_ref[0].reshape(16 * NP, DIM)
    lanes2 = jax.lax.broadcasted_iota(jnp.int32, (4 * NP, 2 * NP), 1)
    laneso = jax.lax.broadcasted_iota(jnp.int32, (4 * NP, 2 * HD), 1)
    z56 = jnp.zeros((NP, HD), f32)
    o1 = jnp.ones((NP, 1), f32)
    zz1 = jnp.zeros((NP, 1), f32)
    sumcols = jnp.concatenate([
        jnp.concatenate([o1, zz1], axis=1),
        jnp.concatenate([zz1, o1], axis=1)], axis=0)
    for b in range(4):
        kf = k_ref[0, b]
        vf = v_ref[0, b]
        for i in range(HEADS // 2):
            sl = slice(i * 2 * HD, (i + 1) * 2 * HD)
            k1 = kf[:, i * 2 * HD:i * 2 * HD + HD]
            k2 = kf[:, i * 2 * HD + HD:(i + 1) * 2 * HD]
            bd_k = jnp.concatenate([
                jnp.concatenate([k1, z56], axis=1),
                jnp.concatenate([z56, k2], axis=1)], axis=0)
            s = jax.lax.dot_general(
                qf[:, sl], bd_k, (((1,), (1,)), ((), ())),
                preferred_element_type=f32)
            s = (s.reshape(16, NP, 2 * NP) + bias_ref[i][None]
                 ).reshape(16 * NP, 2 * NP)
            v1 = vf[:, i * 2 * HD:i * 2 * HD + HD]
            v2 = vf[:, i * 2 * HD + HD:(i + 1) * 2 * HD]
            bd_v = jnp.concatenate([
                jnp.concatenate([v1, z56, sumcols[:NP]], axis=1),
                jnp.concatenate([z56, v2, sumcols[NP:]], axis=1)], axis=0)
            o4 = jnp.zeros((4 * NP, 2 * HD), f32)
            for t in range(4):
                st = s[t * 4 * NP:(t + 1) * 4 * NP]
                m1 = jnp.max(st[:, :NP], axis=-1, keepdims=True)
                m2 = jnp.max(st[:, NP:], axis=-1, keepdims=True)
                e = jnp.exp(st - jnp.where(lanes2 < NP, m1, m2))
                oa = jax.lax.dot_general(
                    e, bd_v, (((1,), (0,)), ((), ())),
                    preferred_element_type=f32)
                rs1 = 1.0 / oa[:, 2 * HD:2 * HD + 1]
                rs2 = 1.0 / oa[:, 2 * HD + 1:2 * HD + 2]
                o4 = o4 + oa[:, :2 * HD] * jnp.where(laneso < HD, rs1, rs2)
            o_ref[:, b, :, sl] = (o4 * 0.25).reshape(4, NP, 2 * HD)


def _mlp_body(a_ref, x_ref, wp_ref, bp_ref, g2_ref, be2_ref,
              w1_ref, b1_ref, w2_ref, b2_ref, o_ref, awb):
    for wl in range(2):
        for wc in range(4):
            for i in range(WS):
                src = (wl * 4 + wc) * NP + i * WS
                dst = wl * 196 + i * 28 + wc * WS
                awb[dst:dst + WS, :] = a_ref[src:src + WS, :]
    z = jax.lax.dot_general(
        awb[...], wp_ref[...], (((1,), (1,)), ((), ())),
        preferred_element_type=jnp.float32) + bp_ref[...]
    x2 = x_ref[...] + z
    mu = jnp.mean(x2, axis=1, keepdims=True)
    var = jnp.mean((x2 - mu) ** 2, axis=1, keepdims=True)
    ln = (x2 - mu) * jax.lax.rsqrt(var + 1e-5) * g2_ref[...] + be2_ref[...]
    h1 = jax.lax.dot_general(
        ln, w1_ref[...], (((1,), (1,)), ((), ())),
        preferred_element_type=jnp.float32) + b1_ref[...]
    h1 = 0.5 * h1 * (1.0 + jax.lax.erf(h1 * (2.0 ** -0.5)))
    y = jax.lax.dot_general(
        h1, w2_ref[...], (((1,), (1,)), ((), ())),
        preferred_element_type=jnp.float32) + b2_ref[...]
    o_ref[...] = x2 + y


def kernel(x, n1g, n1b, Wqkv, bqkv, rpb, Wproj, bproj, n2g, n2b, W1, b1, W2, b2):
    f32 = jnp.float32
    xf = x.reshape(TOK, DIM)

    qkv3 = pl.pallas_call(
        _qkv_body,
        grid=(8,),
        in_specs=[
            pl.BlockSpec((392, DIM), lambda i: (i, 0)),
            pl.BlockSpec((1, DIM), lambda i: (0, 0)),
            pl.BlockSpec((1, DIM), lambda i: (0, 0)),
            pl.BlockSpec((3 * DIM, DIM), lambda i: (0, 0)),
            pl.BlockSpec((1, 3 * DIM), lambda i: (0, 0)),
        ],
        out_specs=pl.BlockSpec((3, 448, DIM), lambda i: (0, i, 0)),
        out_shape=jax.ShapeDtypeStruct((3, TOKP, DIM), f32),
        scratch_shapes=[pltpu.VMEM((448, DIM), f32)],
    )(xf, n1g.reshape(1, DIM), n1b.reshape(1, DIM), Wqkv, bqkv.reshape(1, 3 * DIM))

    q6 = qkv3.reshape(3, 4, 4, 4, NP, DIM)
    kv4 = qkv3.reshape(3, B * NW, NP, DIM)

    rpbg = rpb[_RPI_FLAT].reshape(N, N, HEADS)
    full = jnp.full((HEADS, NP, NP), -1e30, f32).at[:, :N, :N].set(
        rpbg.transpose(2, 0, 1))
    bias2 = jnp.concatenate([full[0::2], full[1::2]], axis=2)

    att = pl.pallas_call(
        _attn_body,
        grid=(4,),
        in_specs=[
            pl.BlockSpec((1, 4, 4, 1, NP, DIM), lambda r: (0, 0, 0, r, 0, 0)),
            pl.BlockSpec((1, 4, NP, DIM), lambda r: (1, r, 0, 0)),
            pl.BlockSpec((1, 4, NP, DIM), lambda r: (2, r, 0, 0)),
            pl.BlockSpec((HEADS // 2, NP, 2 * NP), lambda r: (0, 0, 0)),
        ],
        out_specs=pl.BlockSpec((4, 4, NP, DIM), lambda r: (0, r, 0, 0)),
        out_shape=jax.ShapeDtypeStruct((B, NW, NP, DIM), f32),
    )(q6, kv4, kv4, bias2)

    attf = att.reshape(TOKP, DIM)

    out = pl.pallas_call(
        _mlp_body,
        grid=(8,),
        in_specs=[
            pl.BlockSpec((448, DIM), lambda i: (i, 0)),
            pl.BlockSpec((392, DIM), lambda i: (i, 0)),
            pl.BlockSpec((DIM, DIM), lambda i: (0, 0)),
            pl.BlockSpec((1, DIM), lambda i: (0, 0)),
            pl.BlockSpec((1, DIM), lambda i: (0, 0)),
            pl.BlockSpec((1, DIM), lambda i: (0, 0)),
            pl.BlockSpec((HIDDEN, DIM), lambda i: (0, 0)),
            pl.BlockSpec((1, HIDDEN), lambda i: (0, 0)),
            pl.BlockSpec((DIM, HIDDEN), lambda i: (0, 0)),
            pl.BlockSpec((1, DIM), lambda i: (0, 0)),
        ],
        out_specs=pl.BlockSpec((392, DIM), lambda i: (i, 0)),
        out_shape=jax.ShapeDtypeStruct((TOK, DIM), f32),
        scratch_shapes=[pltpu.VMEM((392, DIM), f32)],
    )(attf, xf, Wproj, bproj.reshape(1, DIM),
      n2g.reshape(1, DIM), n2b.reshape(1, DIM),
      W1, b1.reshape(1, HIDDEN), W2, b2.reshape(1, DIM))

    return out.reshape(B, H * W, DIM)


# speedup vs baseline: 46.0842x; 1.0097x over previous
import jax
import jax.numpy as jnp
import numpy as np
from jax.experimental import pallas as pl
from jax.experimental.pallas import tpu as pltpu

DIM = 384
HEADS = 12
HD = DIM // HEADS
WS = 7
H = 28
W = 28
B = 4
NW = 16
N = WS * WS
NP = 56
HIDDEN = 1536
TOK = B * H * W
TOKP = B * NW * NP
SCALE = HD ** -0.5


def _rel_pos_index(ws):
    coords = np.stack(np.meshgrid(np.arange(ws), np.arange(ws), indexing='ij'))
    cf = coords.reshape(2, -1)
    rel = cf[:, :, None] - cf[:, None, :]
    rel = rel.transpose(1, 2, 0).copy()
    rel[:, :, 0] += ws - 1
    rel[:, :, 1] += ws - 1
    rel[:, :, 0] *= 2 * ws - 1
    return rel.sum(-1)

_RPI_FLAT = np.asarray(_rel_pos_index(WS).reshape(-1), dtype=np.int32)


def _body(x_ref, g1_ref, b1_ref, wqkv_ref, bqkv_ref, bias_ref,
          wp_ref, bp_ref, g2_ref, be2_ref, w1_ref, bm1_ref, w2_ref, bm2_ref,
          o_ref, qkv_s, att_s, xwb, awb, qbuf):
    f32 = jnp.float32
    pid = pl.program_id(0)

    @pl.when(pid < 8)
    def _phase_qkv():
        for wl in range(2):
            for wc in range(4):
                for i in range(WS):
                    dst = (wl * 4 + wc) * NP + i * WS
                    src = wl * 196 + i * 28 + wc * WS
                    xwb[dst:dst + WS, :] = x_ref[src:src + WS, :]
                xwb[(wl * 4 + wc) * NP + N:(wl * 4 + wc) * NP + NP, :] = (
                    jnp.zeros((NP - N, DIM), f32))
        xv = xwb[...]
        mu = jnp.mean(xv, axis=1, keepdims=True)
        var = jnp.mean((xv - mu) ** 2, axis=1, keepdims=True)
        ln = (xv - mu) * jax.lax.rsqrt(var + 1e-5) * g1_ref[...] + b1_ref[...]
        res = jax.lax.dot_general(
            ln, wqkv_ref[...], (((1,), (1,)), ((), ())),
            preferred_element_type=f32) + bqkv_ref[...]
        qkv_s[0, pl.ds(pid * 448, 448), :] = res[:, :DIM] * SCALE
        qkv_s[1, pl.ds(pid * 448, 448), :] = res[:, DIM:2 * DIM]
        qkv_s[2, pl.ds(pid * 448, 448), :] = res[:, 2 * DIM:]

    @pl.when((pid >= 8) & (pid < 12))
    def _phase_attn():
        r = pid - 8
        for t in range(4):
            for a in range(4):
                dst = (t * 4 + a) * NP
                qbuf[dst:dst + NP, :] = qkv_s[0, pl.ds((t * 16 + a * 4 + r) * NP, NP), :]
        qf = qbuf[...]
        lanes2 = jax.lax.broadcasted_iota(jnp.int32, (4 * NP, 2 * NP), 1)
        laneso = jax.lax.broadcasted_iota(jnp.int32, (4 * NP, 2 * HD), 1)
        z56 = jnp.zeros((NP, HD), f32)
        o1 = jnp.ones((NP, 1), f32)
        zz1 = jnp.zeros((NP, 1), f32)
        sumcols = jnp.concatenate([
            jnp.concatenate([o1, zz1], axis=1),
            jnp.concatenate([zz1, o1], axis=1)], axis=0)
        for b in range(4):
            kf = qkv_s[1, pl.ds((4 * r + b) * NP, NP), :]
            vf = qkv_s[2, pl.ds((4 * r + b) * NP, NP), :]
            for i in range(HEADS // 2):
                sl = slice(i * 2 * HD, (i + 1) * 2 * HD)
                k1 = kf[:, i * 2 * HD:i * 2 * HD + HD]
                k2 = kf[:, i * 2 * HD + HD:(i + 1) * 2 * HD]
                bd_k = jnp.concatenate([
                    jnp.concatenate([k1, z56], axis=1),
                    jnp.concatenate([z56, k2], axis=1)], axis=0)
                s = jax.lax.dot_general(
                    qf[:, sl], bd_k, (((1,), (1,)), ((), ())),
                    preferred_element_type=f32)
                s = (s.reshape(16, NP, 2 * NP) + bias_ref[i][None]
                     ).reshape(16 * NP, 2 * NP)
                v1 = vf[:, i * 2 * HD:i * 2 * HD + HD]
                v2 = vf[:, i * 2 * HD + HD:(i + 1) * 2 * HD]
                bd_v = jnp.concatenate([
                    jnp.concatenate([v1, z56, sumcols[:NP]], axis=1),
                    jnp.concatenate([z56, v2, sumcols[NP:]], axis=1)],
                    axis=0)
                o4 = jnp.zeros((4 * NP, 2 * HD), f32)
                for t in range(4):
                    st = s[t * 4 * NP:(t + 1) * 4 * NP]
                    m1 = jnp.max(st[:, :NP], axis=-1, keepdims=True)
                    m2 = jnp.max(st[:, NP:], axis=-1, keepdims=True)
                    e = jnp.exp(st - jnp.where(lanes2 < NP, m1, m2))
                    oa = jax.lax.dot_general(
                        e, bd_v, (((1,), (0,)), ((), ())),
                        preferred_element_type=f32)
                    rs1 = 1.0 / oa[:, 2 * HD:2 * HD + 1]
                    rs2 = 1.0 / oa[:, 2 * HD + 1:2 * HD + 2]
                    o4 = o4 + oa[:, :2 * HD] * jnp.where(laneso < HD, rs1, rs2)
                o4 = (o4 * 0.25).reshape(4, NP, 2 * HD)
                for a in range(4):
                    att_s[pl.ds((a * 16 + 4 * r + b) * NP, NP), sl] = o4[a]

    @pl.when(pid >= 12)
    def _phase_mlp():
        j = pid - 12
        xwb[...] = att_s[pl.ds(j * 448, 448), :]
        for wl in range(2):
            for wc in range(4):
                for i in range(WS):
                    src = (wl * 4 + wc) * NP + i * WS
                    dst = wl * 196 + i * 28 + wc * WS
                    awb[dst:dst + WS, :] = xwb[src:src + WS, :]
        z = jax.lax.dot_general(
            awb[...], wp_ref[...], (((1,), (1,)), ((), ())),
            preferred_element_type=f32) + bp_ref[...]
        x2 = x_ref[...] + z
        mu = jnp.mean(x2, axis=1, keepdims=True)
        var = jnp.mean((x2 - mu) ** 2, axis=1, keepdims=True)
        ln = (x2 - mu) * jax.lax.rsqrt(var + 1e-5) * g2_ref[...] + be2_ref[...]
        h1 = jax.lax.dot_general(
            ln, w1_ref[...], (((1,), (1,)), ((), ())),
            preferred_element_type=f32) + bm1_ref[...]
        h1 = 0.5 * h1 * (1.0 + jax.lax.erf(h1 * (2.0 ** -0.5)))
        y = jax.lax.dot_general(
            h1, w2_ref[...], (((1,), (1,)), ((), ())),
            preferred_element_type=f32) + bm2_ref[...]
        o_ref[...] = x2 + y


def kernel(x, n1g, n1b, Wqkv, bqkv, rpb, Wproj, bproj, n2g, n2b, W1, b1, W2, b2):
    f32 = jnp.float32
    xf = x.reshape(TOK, DIM)

    rpbg = rpb[_RPI_FLAT].reshape(N, N, HEADS)
    full = jnp.full((HEADS, NP, NP), -1e30, f32).at[:, :N, :N].set(
        rpbg.transpose(2, 0, 1))
    bias2 = jnp.concatenate([full[0::2], full[1::2]], axis=2)

    c0 = lambda i: (0, 0)
    out = pl.pallas_call(
        _body,
        grid=(20,),
        in_specs=[
            pl.BlockSpec((392, DIM),
                         lambda g: (jnp.where(g < 8, g,
                                              jnp.clip(g - 12, 0, 7)), 0)),
            pl.BlockSpec((1, DIM), c0),
            pl.BlockSpec((1, DIM), c0),
            pl.BlockSpec((3 * DIM, DIM), c0),
            pl.BlockSpec((1, 3 * DIM), c0),
            pl.BlockSpec((HEADS // 2, NP, 2 * NP), lambda g: (0, 0, 0)),
            pl.BlockSpec((DIM, DIM), c0),
            pl.BlockSpec((1, DIM), c0),
            pl.BlockSpec((1, DIM), c0),
            pl.BlockSpec((1, DIM), c0),
            pl.BlockSpec((HIDDEN, DIM), c0),
            pl.BlockSpec((1, HIDDEN), c0),
            pl.BlockSpec((DIM, HIDDEN), c0),
            pl.BlockSpec((1, DIM), c0),
        ],
        out_specs=pl.BlockSpec((392, DIM),
                               lambda g: (jnp.where(g >= 12, g - 12, 0), 0)),
        out_shape=jax.ShapeDtypeStruct((TOK, DIM), f32),
        scratch_shapes=[
            pltpu.VMEM((3, TOKP, DIM), f32),
            pltpu.VMEM((TOKP, DIM), f32),
            pltpu.VMEM((448, DIM), f32),
            pltpu.VMEM((392, DIM), f32),
            pltpu.VMEM((16 * NP, DIM), f32),
        ],
    )(xf, n1g.reshape(1, DIM), n1b.reshape(1, DIM), Wqkv,
      bqkv.reshape(1, 3 * DIM), bias2, Wproj, bproj.reshape(1, DIM),
      n2g.reshape(1, DIM), n2b.reshape(1, DIM),
      W1, b1.reshape(1, HIDDEN), W2, b2.reshape(1, DIM))

    return out.reshape(B, H * W, DIM)
